# baseline (speedup 1.0000x reference)
import functools

import jax
import jax.numpy as jnp
from jax import lax
from jax.experimental import pallas as pl
from jax.experimental.pallas import tpu as pltpu
from jax.experimental.pallas import tpu_sc as plsc

D_MODEL = 1024
D_FF = 2048
N_EXPERTS = 8
N_TOK = 2048
N_ASN = 2 * N_TOK
BM = 256
MAX_TILES = 24
AP = MAX_TILES * BM
DUMP_BLK = MAX_TILES - 1
NW = 32


def _router_body(x_ref, rw_ref, pos0_ref, pos1_ref, perm_ref, wsort_ref,
                 meta_ref):
    N, E, A = N_TOK, N_EXPERTS, N_ASN
    x = x_ref[...]
    rw = rw_ref[...]
    logits = lax.dot_general(x, rw, (((1,), (1,)), ((), ())),
                             preferred_element_type=jnp.float32)

    idx8 = lax.broadcasted_iota(jnp.int32, (N, E), 1)
    big_neg = jnp.float32(-3.0e38)
    m1 = jnp.max(logits, axis=1, keepdims=True)
    i1 = jnp.min(jnp.where(logits == m1, idx8, E), axis=1, keepdims=True)
    l2 = jnp.where(idx8 == i1, big_neg, logits)
    m2 = jnp.max(l2, axis=1, keepdims=True)
    i2 = jnp.min(jnp.where(l2 == m2, idx8, E), axis=1, keepdims=True)
    d = jnp.exp(m2 - m1)
    w1 = 1.0 / (1.0 + d)
    w2 = d / (1.0 + d)

    oh1 = (idx8 == i1).astype(jnp.float32)
    oh2 = (idx8 == i2).astype(jnp.float32)
    counts = (jnp.sum(oh1, axis=0, keepdims=True)
              + jnp.sum(oh2, axis=0, keepdims=True))
    cpad = (((counts.astype(jnp.int32) + (BM - 1)) // BM) * BM).astype(
        jnp.float32)
    lt8 = (lax.broadcasted_iota(jnp.int32, (E, E), 0)
           < lax.broadcasted_iota(jnp.int32, (E, E), 1)).astype(jnp.float32)
    offs = jnp.dot(cpad, lt8, preferred_element_type=jnp.float32)

    BL = 1024
    ltb = (lax.broadcasted_iota(jnp.int32, (BL, BL), 0)
           > lax.broadcasted_iota(jnp.int32, (BL, BL), 1)).astype(jnp.float32)
    base = jnp.zeros((1, E), jnp.float32)
    pos_blocks = []
    oh_blocks = [oh1[:BL], oh1[BL:], oh2[:BL], oh2[BL:]]
    for ohb in oh_blocks:
        rb = jnp.dot(ltb, ohb, preferred_element_type=jnp.float32) + base
        base = base + jnp.sum(ohb, axis=0, keepdims=True)
        pos_blocks.append(jnp.sum((rb + offs) * ohb, axis=1, keepdims=True))
    pos = jnp.concatenate(pos_blocks, axis=0)
    pos0_ref[...] = pos[:N].astype(jnp.int32)
    pos1_ref[...] = pos[N:].astype(jnp.int32)

    tok = lax.broadcasted_iota(jnp.int32, (N, 1), 0).astype(jnp.float32)
    tok_a = jnp.concatenate([tok, tok], axis=0)
    w_a = jnp.concatenate([w1, w2], axis=0)
    for pb in range(AP // 256):
        pids = (jnp.float32(pb * 256)
                + lax.broadcasted_iota(jnp.int32, (1, 256), 1).astype(
                    jnp.float32))
        m = pos == pids
        r, c = pb // 2, (pb % 2) * 256
        perm_ref[r:r + 1, c:c + 256] = jnp.sum(
            jnp.where(m, tok_a, 0.0), axis=0, keepdims=True).astype(jnp.int32)
        wsort_ref[r:r + 1, c:c + 256] = jnp.sum(
            jnp.where(m, w_a, 0.0), axis=0, keepdims=True)

    counts_i = counts.astype(jnp.int32)
    nt = (counts_i + (BM - 1)) // BM
    nt_f = nt.astype(jnp.float32)
    le8 = (lax.broadcasted_iota(jnp.int32, (E, E), 0)
           <= lax.broadcasted_iota(jnp.int32, (E, E), 1)).astype(jnp.float32)
    cumnt = jnp.dot(nt_f, le8, preferred_element_type=jnp.float32)
    tstart = cumnt - nt_f
    tid = lax.broadcasted_iota(jnp.int32, (MAX_TILES, 1), 0).astype(
        jnp.float32)
    te = jnp.sum((cumnt <= tid).astype(jnp.float32), axis=1, keepdims=True)
    te = jnp.minimum(te, jnp.float32(E - 1))
    idx8t = lax.broadcasted_iota(jnp.int32, (MAX_TILES, E), 1).astype(
        jnp.float32)
    oh_te = (idx8t == te).astype(jnp.float32)
    tstart_te = jnp.sum(oh_te * tstart, axis=1, keepdims=True)
    offs_te = jnp.sum(oh_te * offs, axis=1, keepdims=True)
    cnt_te = jnp.sum(oh_te * counts, axis=1, keepdims=True)
    j = tid - tstart_te
    rsb = offs_te / float(BM) + j
    nv = jnp.clip(cnt_te - j * BM, 0.0, float(BM))
    total = jnp.sum(nt_f)
    nv = jnp.where(tid < total, nv, 0.0)
    rsb = jnp.where(tid < total, rsb, float(DUMP_BLK))
    meta_ref[...] = jnp.concatenate([te, rsb, nv, tid],
                                    axis=1).astype(jnp.int32)


def _run_router(x_flat, router_w):
    return pl.pallas_call(
        _router_body,
        out_shape=(
            jax.ShapeDtypeStruct((N_TOK, 1), jnp.int32),
            jax.ShapeDtypeStruct((N_TOK, 1), jnp.int32),
            jax.ShapeDtypeStruct((AP // 512, 512), jnp.int32),
            jax.ShapeDtypeStruct((AP // 512, 512), jnp.float32),
            jax.ShapeDtypeStruct((MAX_TILES, 4), jnp.int32),
        ),
        compiler_params=pltpu.CompilerParams(
            vmem_limit_bytes=100 * 1024 * 1024),
    )(x_flat, router_w)


DH = D_MODEL // 2


def _sc_gather_rows(x_i32, perm_pad):
    rows_w = AP // NW
    nch = rows_w // 64
    mesh = plsc.VectorSubcoreMesh(core_axis_name="c", subcore_axis_name="s")

    @functools.partial(
        pl.kernel,
        out_type=jax.ShapeDtypeStruct((AP, DH), jnp.int32),
        mesh=mesh,
        scratch_types=[
            pltpu.VMEM((rows_w,), jnp.int32),
            pltpu.VMEM((64, DH), jnp.int32),
            pltpu.VMEM((64, DH), jnp.int32),
            pltpu.VMEM((64, DH), jnp.int32),
            pltpu.SemaphoreType.DMA,
        ],
    )
    def gather_k(x_hbm, idx_hbm, out_hbm, idx_v, b0, b1, b2, sem):
        wid = lax.axis_index("s") * 2 + lax.axis_index("c")
        base = wid * rows_w
        bufs = [b0, b1, b2]
        pltpu.sync_copy(idx_hbm.at[pl.ds(base, rows_w)], idx_v)
        cps = [pltpu.async_copy(x_hbm.at[idx_v.at[pl.ds(c * 64, 64)]],
                                bufs[c], sem) for c in range(nch)]
        for c in range(nch):
            cps[c].wait()
            pltpu.sync_copy(bufs[c], out_hbm.at[pl.ds(base + c * 64, 64)])

    return gather_k(x_i32, perm_pad)


def _mlp_body(meta_ref, x_ref, w_ref, wgu_ref, wd_ref, o_ref):
    t = pl.program_id(0)
    nv = meta_ref[t, 2]

    @pl.when(nv > 0)
    def _():
        xb = x_ref[...]
        gu = lax.dot_general(xb, wgu_ref[0], (((1,), (0,)), ((), ())),
                             preferred_element_type=jnp.float32)
        g = gu[:, :D_FF]
        u = gu[:, D_FF:]
        inter = (g * (1.0 / (1.0 + jnp.exp(-g)))) * u
        interb = (inter * w_ref[...]).astype(jnp.bfloat16)
        o_ref[...] = lax.dot_general(
            interb, wd_ref[0], (((1,), (0,)), ((), ())),
            preferred_element_type=jnp.float32).astype(jnp.bfloat16)


def _run_mlp(meta, x_sorted, w_pad, wgu_bf, wd_bf):
    grid_spec = pltpu.PrefetchScalarGridSpec(
        num_scalar_prefetch=1,
        grid=(MAX_TILES,),
        in_specs=[
            pl.BlockSpec((BM, D_MODEL), lambda t, m: (m[t, 1], 0)),
            pl.BlockSpec((BM, 1), lambda t, m: (m[t, 1], 0)),
            pl.BlockSpec((1, D_MODEL, 2 * D_FF), lambda t, m: (m[t, 0], 0, 0)),
            pl.BlockSpec((1, D_FF, D_MODEL), lambda t, m: (m[t, 0], 0, 0)),
        ],
        out_specs=pl.BlockSpec((BM, D_MODEL), lambda t, m: (m[t, 1], 0)),
    )
    return pl.pallas_call(
        _mlp_body,
        grid_spec=grid_spec,
        out_shape=jax.ShapeDtypeStruct((AP, D_MODEL), jnp.bfloat16),
    )(meta, x_sorted, w_pad, wgu_bf, wd_bf)


def _sc_combine(y_i32, pos0, pos1):
    tok_w = N_TOK // NW
    mesh = plsc.VectorSubcoreMesh(core_axis_name="c", subcore_axis_name="s")

    @functools.partial(
        pl.kernel,
        out_type=jax.ShapeDtypeStruct((N_TOK, DH), jnp.int32),
        mesh=mesh,
        scratch_types=[
            pltpu.VMEM((tok_w,), jnp.int32),
            pltpu.VMEM((tok_w,), jnp.int32),
            pltpu.VMEM((tok_w, DH), jnp.int32),
            pltpu.VMEM((tok_w, DH), jnp.int32),
            pltpu.SemaphoreType.DMA,
        ],
        compiler_params=pltpu.CompilerParams(needs_layout_passes=False),
    )
    def combine_k(y_hbm, p0_hbm, p1_hbm, out_hbm, i0_v, i1_v, b0, b1, sem):
        wid = lax.axis_index("s") * 2 + lax.axis_index("c")
        base = wid * tok_w
        pltpu.sync_copy(p0_hbm.at[pl.ds(base, tok_w)], i0_v)
        pltpu.sync_copy(p1_hbm.at[pl.ds(base, tok_w)], i1_v)
        cp0 = pltpu.async_copy(y_hbm.at[i0_v], b0, sem)
        cp1 = pltpu.async_copy(y_hbm.at[i1_v], b1, sem)
        cp0.wait()
        cp1.wait()

        def add_row(r, carry):
            for jj in range(DH // 16):
                sl = pl.ds(jj * 16, 16)
                a = plsc.bitcast(b0[r, sl], jnp.bfloat16)
                b = plsc.bitcast(b1[r, sl], jnp.bfloat16)
                b0[r, sl] = plsc.bitcast(a + b, jnp.int32)
            return carry

        lax.fori_loop(0, tok_w, add_row, 0)
        pltpu.sync_copy(b0, out_hbm.at[pl.ds(base, tok_w)])

    return combine_k(y_i32, pos0, pos1)


def kernel(x, router_w, w_gate_up, w_down):
    Bb, Tt, D = x.shape
    x_flat = x.reshape(-1, D)

    pos0, pos1, perm, wsort, meta = _run_router(x_flat, router_w)

    perm_pad = perm.reshape(AP)
    w_pad = wsort.reshape(AP, 1)

    x_i32 = lax.bitcast_convert_type(
        x_flat.astype(jnp.bfloat16).reshape(N_TOK, DH, 2), jnp.int32)
    xs_i32 = _sc_gather_rows(x_i32, perm_pad)
    x_sorted = lax.bitcast_convert_type(
        xs_i32, jnp.bfloat16).reshape(AP, D_MODEL)

    wgu_bf = w_gate_up.astype(jnp.bfloat16)
    wd_bf = w_down.astype(jnp.bfloat16)
    y_sorted = _run_mlp(meta, x_sorted, w_pad, wgu_bf, wd_bf)

    y_i32 = lax.bitcast_convert_type(y_sorted.reshape(AP, DH, 2), jnp.int32)
    out_i32 = _sc_combine(y_i32, pos0.reshape(N_TOK), pos1.reshape(N_TOK))
    out = lax.bitcast_convert_type(out_i32, jnp.bfloat16).reshape(
        N_TOK, D_MODEL).astype(jnp.float32)
    return out.reshape(Bb, Tt, D)


# speedup vs baseline: 1.8959x; 1.8959x over previous
import functools

import jax
import jax.numpy as jnp
from jax import lax
from jax.experimental import pallas as pl
from jax.experimental.pallas import tpu as pltpu
from jax.experimental.pallas import tpu_sc as plsc

D_MODEL = 1024
D_FF = 2048
N_EXPERTS = 8
N_TOK = 2048
N_ASN = 2 * N_TOK
BM = 256
MAX_TILES = 24
AP = MAX_TILES * BM
DUMP_BLK = MAX_TILES - 1
NW = 32


def _router_body(x_ref, rw_ref, pos0_ref, pos1_ref, perm_ref, wsort_ref,
                 meta_ref):
    N, E, A = N_TOK, N_EXPERTS, N_ASN
    x = x_ref[...]
    rw = rw_ref[...]
    logits = lax.dot_general(x, rw, (((1,), (1,)), ((), ())),
                             preferred_element_type=jnp.float32)

    idx8 = lax.broadcasted_iota(jnp.int32, (N, E), 1)
    big_neg = jnp.float32(-3.0e38)
    m1 = jnp.max(logits, axis=1, keepdims=True)
    i1 = jnp.min(jnp.where(logits == m1, idx8, E), axis=1, keepdims=True)
    l2 = jnp.where(idx8 == i1, big_neg, logits)
    m2 = jnp.max(l2, axis=1, keepdims=True)
    i2 = jnp.min(jnp.where(l2 == m2, idx8, E), axis=1, keepdims=True)
    d = jnp.exp(m2 - m1)
    w1 = 1.0 / (1.0 + d)
    w2 = d / (1.0 + d)

    oh1 = (idx8 == i1).astype(jnp.float32)
    oh2 = (idx8 == i2).astype(jnp.float32)
    counts = (jnp.sum(oh1, axis=0, keepdims=True)
              + jnp.sum(oh2, axis=0, keepdims=True))
    cpad = (((counts.astype(jnp.int32) + (BM - 1)) // BM) * BM).astype(
        jnp.float32)
    lt8 = (lax.broadcasted_iota(jnp.int32, (E, E), 0)
           < lax.broadcasted_iota(jnp.int32, (E, E), 1)).astype(jnp.float32)
    offs = jnp.dot(cpad, lt8, preferred_element_type=jnp.float32)

    BL = 1024
    ltb = (lax.broadcasted_iota(jnp.int32, (BL, BL), 0)
           > lax.broadcasted_iota(jnp.int32, (BL, BL), 1)).astype(jnp.float32)
    base = jnp.zeros((1, E), jnp.float32)
    pos_blocks = []
    oh_blocks = [oh1[:BL], oh1[BL:], oh2[:BL], oh2[BL:]]
    for ohb in oh_blocks:
        rb = jnp.dot(ltb, ohb, preferred_element_type=jnp.float32) + base
        base = base + jnp.sum(ohb, axis=0, keepdims=True)
        pos_blocks.append(jnp.sum((rb + offs) * ohb, axis=1, keepdims=True))
    pos = jnp.concatenate(pos_blocks, axis=0)
    pos0_ref[...] = pos[:N].astype(jnp.int32)
    pos1_ref[...] = pos[N:].astype(jnp.int32)

    tok = lax.broadcasted_iota(jnp.int32, (N, 1), 0).astype(jnp.float32)
    tok_a = jnp.concatenate([tok, tok], axis=0)
    w_a = jnp.concatenate([w1, w2], axis=0)
    for pb in range(AP // 256):
        pids = (jnp.float32(pb * 256)
                + lax.broadcasted_iota(jnp.int32, (1, 256), 1).astype(
                    jnp.float32))
        m = pos == pids
        r, c = pb // 2, (pb % 2) * 256
        perm_ref[r:r + 1, c:c + 256] = jnp.sum(
            jnp.where(m, tok_a, 0.0), axis=0, keepdims=True).astype(jnp.int32)
        wsort_ref[r:r + 1, c:c + 256] = jnp.sum(
            jnp.where(m, w_a, 0.0), axis=0, keepdims=True)

    counts_i = counts.astype(jnp.int32)
    nt = (counts_i + (BM - 1)) // BM
    nt_f = nt.astype(jnp.float32)
    le8 = (lax.broadcasted_iota(jnp.int32, (E, E), 0)
           <= lax.broadcasted_iota(jnp.int32, (E, E), 1)).astype(jnp.float32)
    cumnt = jnp.dot(nt_f, le8, preferred_element_type=jnp.float32)
    tstart = cumnt - nt_f
    tid = lax.broadcasted_iota(jnp.int32, (MAX_TILES, 1), 0).astype(
        jnp.float32)
    te = jnp.sum((cumnt <= tid).astype(jnp.float32), axis=1, keepdims=True)
    te = jnp.minimum(te, jnp.float32(E - 1))
    idx8t = lax.broadcasted_iota(jnp.int32, (MAX_TILES, E), 1).astype(
        jnp.float32)
    oh_te = (idx8t == te).astype(jnp.float32)
    tstart_te = jnp.sum(oh_te * tstart, axis=1, keepdims=True)
    offs_te = jnp.sum(oh_te * offs, axis=1, keepdims=True)
    cnt_te = jnp.sum(oh_te * counts, axis=1, keepdims=True)
    j = tid - tstart_te
    rsb = offs_te / float(BM) + j
    nv = jnp.clip(cnt_te - j * BM, 0.0, float(BM))
    total = jnp.sum(nt_f)
    nv = jnp.where(tid < total, nv, 0.0)
    rsb = jnp.where(tid < total, rsb, float(DUMP_BLK))
    meta_ref[...] = jnp.concatenate([te, rsb, nv, tid],
                                    axis=1).astype(jnp.int32)


def _run_router(x_flat, router_w):
    return pl.pallas_call(
        _router_body,
        out_shape=(
            jax.ShapeDtypeStruct((N_TOK, 1), jnp.int32),
            jax.ShapeDtypeStruct((N_TOK, 1), jnp.int32),
            jax.ShapeDtypeStruct((AP // 512, 512), jnp.int32),
            jax.ShapeDtypeStruct((AP // 512, 512), jnp.float32),
            jax.ShapeDtypeStruct((MAX_TILES, 4), jnp.int32),
        ),
        compiler_params=pltpu.CompilerParams(
            vmem_limit_bytes=100 * 1024 * 1024),
    )(x_flat, router_w)


def _sc_gather_rows(x_flat, perm_pad):
    rows_w = AP // NW
    nch = rows_w // 32
    mesh = plsc.VectorSubcoreMesh(core_axis_name="c", subcore_axis_name="s")

    @functools.partial(
        pl.kernel,
        out_type=jax.ShapeDtypeStruct((AP, D_MODEL), jnp.float32),
        mesh=mesh,
        scratch_types=[
            pltpu.VMEM((rows_w,), jnp.int32),
            pltpu.VMEM((32, D_MODEL), jnp.float32),
            pltpu.VMEM((32, D_MODEL), jnp.float32),
            pltpu.VMEM((32, D_MODEL), jnp.float32),
            pltpu.SemaphoreType.DMA,
        ],
    )
    def gather_k(x_hbm, idx_hbm, out_hbm, idx_v, b0, b1, b2, sem):
        wid = lax.axis_index("s") * 2 + lax.axis_index("c")
        base = wid * rows_w
        bufs = [b0, b1, b2]
        pltpu.sync_copy(idx_hbm.at[pl.ds(base, rows_w)], idx_v)
        cps = [None] * nch
        for c in range(3):
            cps[c] = pltpu.async_copy(
                x_hbm.at[idx_v.at[pl.ds(c * 32, 32)]], bufs[c], sem)
        for c in range(nch):
            cps[c].wait()
            pltpu.sync_copy(bufs[c % 3], out_hbm.at[pl.ds(base + c * 32, 32)])
            if c + 3 < nch:
                cps[c + 3] = pltpu.async_copy(
                    x_hbm.at[idx_v.at[pl.ds((c + 3) * 32, 32)]], bufs[c % 3],
                    sem)

    return gather_k(x_flat, perm_pad)


def _mlp_body(meta_ref, x_ref, w_ref, wgu_ref, wd_ref, o_ref):
    t = pl.program_id(0)
    nv = meta_ref[t, 2]

    @pl.when(nv > 0)
    def _():
        xb = x_ref[...].astype(jnp.bfloat16)
        gu = lax.dot_general(xb, wgu_ref[0], (((1,), (0,)), ((), ())),
                             preferred_element_type=jnp.float32)
        g = gu[:, :D_FF]
        u = gu[:, D_FF:]
        inter = (g * (1.0 / (1.0 + jnp.exp(-g)))) * u
        interb = (inter * w_ref[...]).astype(jnp.bfloat16)
        o_ref[...] = lax.dot_general(
            interb, wd_ref[0], (((1,), (0,)), ((), ())),
            preferred_element_type=jnp.float32)


def _run_mlp(meta, x_sorted, w_pad, wgu_bf, wd_bf):
    grid_spec = pltpu.PrefetchScalarGridSpec(
        num_scalar_prefetch=1,
        grid=(MAX_TILES,),
        in_specs=[
            pl.BlockSpec((BM, D_MODEL), lambda t, m: (m[t, 1], 0)),
            pl.BlockSpec((BM, 1), lambda t, m: (m[t, 1], 0)),
            pl.BlockSpec((1, D_MODEL, 2 * D_FF), lambda t, m: (m[t, 0], 0, 0)),
            pl.BlockSpec((1, D_FF, D_MODEL), lambda t, m: (m[t, 0], 0, 0)),
        ],
        out_specs=pl.BlockSpec((BM, D_MODEL), lambda t, m: (m[t, 1], 0)),
    )
    return pl.pallas_call(
        _mlp_body,
        grid_spec=grid_spec,
        out_shape=jax.ShapeDtypeStruct((AP, D_MODEL), jnp.float32),
    )(meta, x_sorted, w_pad, wgu_bf, wd_bf)


def _sc_gather_pairs(y_sorted, pos0, pos1):
    tok_w = N_TOK // NW
    mesh = plsc.VectorSubcoreMesh(core_axis_name="c", subcore_axis_name="s")

    @functools.partial(
        pl.kernel,
        out_type=jax.ShapeDtypeStruct((2, N_TOK, D_MODEL), jnp.float32),
        mesh=mesh,
        scratch_types=[
            pltpu.VMEM((tok_w,), jnp.int32),
            pltpu.VMEM((tok_w,), jnp.int32),
            pltpu.VMEM((32, D_MODEL), jnp.float32),
            pltpu.VMEM((32, D_MODEL), jnp.float32),
            pltpu.SemaphoreType.DMA,
        ],
    )
    def pairs_k(y_hbm, p0_hbm, p1_hbm, out_hbm, i0_v, i1_v, b0, b1, sem):
        wid = lax.axis_index("s") * 2 + lax.axis_index("c")
        base = wid * tok_w
        pltpu.sync_copy(p0_hbm.at[pl.ds(base, tok_w)], i0_v)
        pltpu.sync_copy(p1_hbm.at[pl.ds(base, tok_w)], i1_v)
        for c in range(tok_w // 32):
            cp0 = pltpu.async_copy(
                y_hbm.at[i0_v.at[pl.ds(c * 32, 32)]], b0, sem)
            cp1 = pltpu.async_copy(
                y_hbm.at[i1_v.at[pl.ds(c * 32, 32)]], b1, sem)
            cp0.wait()
            pltpu.sync_copy(b0, out_hbm.at[0, pl.ds(base + c * 32, 32)])
            cp1.wait()
            pltpu.sync_copy(b1, out_hbm.at[1, pl.ds(base + c * 32, 32)])

    return pairs_k(y_sorted, pos0, pos1)


def _add_body(a_ref, b_ref, o_ref):
    o_ref[...] = a_ref[0] + b_ref[0]


def _run_pair_add(yp):
    return pl.pallas_call(
        _add_body,
        grid=(N_TOK // BM,),
        in_specs=[
            pl.BlockSpec((1, BM, D_MODEL), lambda m: (0, m, 0)),
            pl.BlockSpec((1, BM, D_MODEL), lambda m: (1, m, 0)),
        ],
        out_specs=pl.BlockSpec((BM, D_MODEL), lambda m: (m, 0)),
        out_shape=jax.ShapeDtypeStruct((N_TOK, D_MODEL), jnp.float32),
    )(yp, yp)


def kernel(x, router_w, w_gate_up, w_down):
    Bb, Tt, D = x.shape
    x_flat = x.reshape(-1, D)

    pos0, pos1, perm, wsort, meta = _run_router(x_flat, router_w)

    perm_pad = perm.reshape(AP)
    w_pad = wsort.reshape(AP, 1)

    x_sorted = _sc_gather_rows(x_flat, perm_pad)

    wgu_bf = w_gate_up.astype(jnp.bfloat16)
    wd_bf = w_down.astype(jnp.bfloat16)
    y_sorted = _run_mlp(meta, x_sorted, w_pad, wgu_bf, wd_bf)

    yp = _sc_gather_pairs(y_sorted, pos0.reshape(N_TOK), pos1.reshape(N_TOK))
    out = _run_pair_add(yp)
    return out.reshape(Bb, Tt, D)


# speedup vs baseline: 2.7935x; 1.4734x over previous
import functools

import jax
import jax.numpy as jnp
from jax import lax
from jax.experimental import pallas as pl
from jax.experimental.pallas import tpu as pltpu
from jax.experimental.pallas import tpu_sc as plsc

D_MODEL = 1024
D_FF = 2048
N_EXPERTS = 8
N_TOK = 2048
N_ASN = 2 * N_TOK
BM = 256
MAX_TILES = 24
AP = MAX_TILES * BM
DUMP_BLK = MAX_TILES - 1
NW = 32


def _router_body(x_ref, rw_ref, pos0_ref, pos1_ref, perm_ref, wsort_ref,
                 meta_ref):
    N, E, A = N_TOK, N_EXPERTS, N_ASN
    x = x_ref[...]
    rw = rw_ref[...]
    logits = lax.dot_general(x, rw, (((1,), (1,)), ((), ())),
                             preferred_element_type=jnp.float32)

    idx8 = lax.broadcasted_iota(jnp.int32, (N, E), 1)
    big_neg = jnp.float32(-3.0e38)
    m1 = jnp.max(logits, axis=1, keepdims=True)
    i1 = jnp.min(jnp.where(logits == m1, idx8, E), axis=1, keepdims=True)
    l2 = jnp.where(idx8 == i1, big_neg, logits)
    m2 = jnp.max(l2, axis=1, keepdims=True)
    i2 = jnp.min(jnp.where(l2 == m2, idx8, E), axis=1, keepdims=True)
    d = jnp.exp(m2 - m1)
    w1 = 1.0 / (1.0 + d)
    w2 = d / (1.0 + d)

    oh1 = (idx8 == i1).astype(jnp.float32)
    oh2 = (idx8 == i2).astype(jnp.float32)
    counts = (jnp.sum(oh1, axis=0, keepdims=True)
              + jnp.sum(oh2, axis=0, keepdims=True))
    cpad = (((counts.astype(jnp.int32) + (BM - 1)) // BM) * BM).astype(
        jnp.float32)
    lt8 = (lax.broadcasted_iota(jnp.int32, (E, E), 0)
           < lax.broadcasted_iota(jnp.int32, (E, E), 1)).astype(jnp.float32)
    offs = jnp.dot(cpad, lt8, preferred_element_type=jnp.float32)

    BL = 1024
    ltb = (lax.broadcasted_iota(jnp.int32, (BL, BL), 0)
           > lax.broadcasted_iota(jnp.int32, (BL, BL), 1)).astype(jnp.float32)
    base = jnp.zeros((1, E), jnp.float32)
    pos_blocks = []
    oh_blocks = [oh1[:BL], oh1[BL:], oh2[:BL], oh2[BL:]]
    for ohb in oh_blocks:
        rb = jnp.dot(ltb, ohb, preferred_element_type=jnp.float32) + base
        base = base + jnp.sum(ohb, axis=0, keepdims=True)
        pos_blocks.append(jnp.sum((rb + offs) * ohb, axis=1, keepdims=True))
    pos = jnp.concatenate(pos_blocks, axis=0)
    pos0_ref[...] = pos[:N].astype(jnp.int32)
    pos1_ref[...] = pos[N:].astype(jnp.int32)

    tok = lax.broadcasted_iota(jnp.int32, (N, 1), 0).astype(jnp.float32)
    tok_a = jnp.concatenate([tok, tok], axis=0)
    w_a = jnp.concatenate([w1, w2], axis=0)
    for pb in range(AP // 256):
        iota256 = lax.broadcasted_iota(jnp.int32, (1, 256), 1).astype(
            jnp.float32)
        pids = jnp.float32(pb * 256) + iota256
        m = pos == pids
        r, c = pb // 2, (pb % 2) * 256
        p1 = jnp.sum(jnp.where(m, tok_a + 1.0, 0.0), axis=0, keepdims=True)
        fill = jnp.float32((pb % 8) * 256) + iota256
        perm_ref[r:r + 1, c:c + 256] = jnp.where(
            p1 == 0.0, fill, p1 - 1.0).astype(jnp.int32)
        wsort_ref[r:r + 1, c:c + 256] = jnp.sum(
            jnp.where(m, w_a, 0.0), axis=0, keepdims=True)

    counts_i = counts.astype(jnp.int32)
    nt = (counts_i + (BM - 1)) // BM
    nt_f = nt.astype(jnp.float32)
    le8 = (lax.broadcasted_iota(jnp.int32, (E, E), 0)
           <= lax.broadcasted_iota(jnp.int32, (E, E), 1)).astype(jnp.float32)
    cumnt = jnp.dot(nt_f, le8, preferred_element_type=jnp.float32)
    tstart = cumnt - nt_f
    tid = lax.broadcasted_iota(jnp.int32, (MAX_TILES, 1), 0).astype(
        jnp.float32)
    te = jnp.sum((cumnt <= tid).astype(jnp.float32), axis=1, keepdims=True)
    te = jnp.minimum(te, jnp.float32(E - 1))
    idx8t = lax.broadcasted_iota(jnp.int32, (MAX_TILES, E), 1).astype(
        jnp.float32)
    oh_te = (idx8t == te).astype(jnp.float32)
    tstart_te = jnp.sum(oh_te * tstart, axis=1, keepdims=True)
    offs_te = jnp.sum(oh_te * offs, axis=1, keepdims=True)
    cnt_te = jnp.sum(oh_te * counts, axis=1, keepdims=True)
    j = tid - tstart_te
    rsb = offs_te / float(BM) + j
    nv = jnp.clip(cnt_te - j * BM, 0.0, float(BM))
    total = jnp.sum(nt_f)
    nv = jnp.where(tid < total, nv, 0.0)
    rsb = jnp.where(tid < total, rsb, float(DUMP_BLK))
    meta_ref[...] = jnp.concatenate([te, rsb, nv, tid],
                                    axis=1).astype(jnp.int32)


def _run_router(x_flat, router_w):
    return pl.pallas_call(
        _router_body,
        out_shape=(
            jax.ShapeDtypeStruct((N_TOK, 1), jnp.int32),
            jax.ShapeDtypeStruct((N_TOK, 1), jnp.int32),
            jax.ShapeDtypeStruct((AP // 512, 512), jnp.int32),
            jax.ShapeDtypeStruct((AP // 512, 512), jnp.float32),
            jax.ShapeDtypeStruct((MAX_TILES, 4), jnp.int32),
        ),
        compiler_params=pltpu.CompilerParams(
            vmem_limit_bytes=100 * 1024 * 1024),
    )(x_flat, router_w)


def _sc_gather_rows(x_flat, perm_pad):
    rows_w = AP // NW
    nch = rows_w // 32
    mesh = plsc.VectorSubcoreMesh(core_axis_name="c", subcore_axis_name="s")

    @functools.partial(
        pl.kernel,
        out_type=jax.ShapeDtypeStruct((AP, D_MODEL), jnp.float32),
        mesh=mesh,
        scratch_types=[
            pltpu.VMEM((rows_w,), jnp.int32),
            pltpu.VMEM((32, D_MODEL), jnp.float32),
            pltpu.VMEM((32, D_MODEL), jnp.float32),
            pltpu.VMEM((32, D_MODEL), jnp.float32),
            pltpu.SemaphoreType.DMA,
        ],
    )
    def gather_k(x_hbm, idx_hbm, out_hbm, idx_v, b0, b1, b2, sem):
        wid = lax.axis_index("s") * 2 + lax.axis_index("c")
        base = wid * rows_w
        bufs = [b0, b1, b2]
        pltpu.sync_copy(idx_hbm.at[pl.ds(base, rows_w)], idx_v)
        cps = [None] * nch
        for c in range(3):
            cps[c] = pltpu.async_copy(
                x_hbm.at[idx_v.at[pl.ds(c * 32, 32)]], bufs[c], sem)
        for c in range(nch):
            cps[c].wait()
            pltpu.sync_copy(bufs[c % 3], out_hbm.at[pl.ds(base + c * 32, 32)])
            if c + 3 < nch:
                cps[c + 3] = pltpu.async_copy(
                    x_hbm.at[idx_v.at[pl.ds((c + 3) * 32, 32)]], bufs[c % 3],
                    sem)

    return gather_k(x_flat, perm_pad)


def _mlp_body(meta_ref, x_ref, w_ref, wgu_ref, wd_ref, o_ref):
    t = pl.program_id(0)
    nv = meta_ref[t, 2]

    @pl.when(nv > 0)
    def _():
        xb = x_ref[...].astype(jnp.bfloat16)
        gu = lax.dot_general(xb, wgu_ref[0], (((1,), (0,)), ((), ())),
                             preferred_element_type=jnp.float32)
        g = gu[:, :D_FF]
        u = gu[:, D_FF:]
        inter = (g * (1.0 / (1.0 + jnp.exp(-g)))) * u
        interb = (inter * w_ref[...]).astype(jnp.bfloat16)
        o_ref[...] = lax.dot_general(
            interb, wd_ref[0].astype(jnp.bfloat16), (((1,), (0,)), ((), ())),
            preferred_element_type=jnp.float32)


def _run_mlp(meta, x_sorted, w_pad, wgu_bf, wd_bf):
    grid_spec = pltpu.PrefetchScalarGridSpec(
        num_scalar_prefetch=1,
        grid=(MAX_TILES,),
        in_specs=[
            pl.BlockSpec((BM, D_MODEL), lambda t, m: (m[t, 1], 0)),
            pl.BlockSpec((BM, 1), lambda t, m: (m[t, 1], 0)),
            pl.BlockSpec((1, D_MODEL, 2 * D_FF), lambda t, m: (m[t, 0], 0, 0)),
            pl.BlockSpec((1, D_FF, D_MODEL), lambda t, m: (m[t, 0], 0, 0)),
        ],
        out_specs=pl.BlockSpec((BM, D_MODEL), lambda t, m: (m[t, 1], 0)),
    )
    return pl.pallas_call(
        _mlp_body,
        grid_spec=grid_spec,
        out_shape=jax.ShapeDtypeStruct((AP, D_MODEL), jnp.float32),
    )(meta, x_sorted, w_pad, wgu_bf, wd_bf)


def _sc_gather_pairs(y_sorted, pos0, pos1):
    tok_w = N_TOK // NW
    mesh = plsc.VectorSubcoreMesh(core_axis_name="c", subcore_axis_name="s")

    @functools.partial(
        pl.kernel,
        out_type=jax.ShapeDtypeStruct((2, N_TOK, D_MODEL), jnp.float32),
        mesh=mesh,
        scratch_types=[
            pltpu.VMEM((tok_w,), jnp.int32),
            pltpu.VMEM((tok_w,), jnp.int32),
            pltpu.VMEM((32, D_MODEL), jnp.float32),
            pltpu.VMEM((32, D_MODEL), jnp.float32),
            pltpu.SemaphoreType.DMA,
        ],
    )
    def pairs_k(y_hbm, p0_hbm, p1_hbm, out_hbm, i0_v, i1_v, b0, b1, sem):
        wid = lax.axis_index("s") * 2 + lax.axis_index("c")
        base = wid * tok_w
        pltpu.sync_copy(p0_hbm.at[pl.ds(base, tok_w)], i0_v)
        pltpu.sync_copy(p1_hbm.at[pl.ds(base, tok_w)], i1_v)
        for c in range(tok_w // 32):
            cp0 = pltpu.async_copy(
                y_hbm.at[i0_v.at[pl.ds(c * 32, 32)]], b0, sem)
            cp1 = pltpu.async_copy(
                y_hbm.at[i1_v.at[pl.ds(c * 32, 32)]], b1, sem)
            cp0.wait()
            pltpu.sync_copy(b0, out_hbm.at[0, pl.ds(base + c * 32, 32)])
            cp1.wait()
            pltpu.sync_copy(b1, out_hbm.at[1, pl.ds(base + c * 32, 32)])

    return pairs_k(y_sorted, pos0, pos1)


def _add_body(a_ref, b_ref, o_ref):
    o_ref[...] = a_ref[0] + b_ref[0]


def _run_pair_add(yp):
    return pl.pallas_call(
        _add_body,
        grid=(N_TOK // BM,),
        in_specs=[
            pl.BlockSpec((1, BM, D_MODEL), lambda m: (0, m, 0)),
            pl.BlockSpec((1, BM, D_MODEL), lambda m: (1, m, 0)),
        ],
        out_specs=pl.BlockSpec((BM, D_MODEL), lambda m: (m, 0)),
        out_shape=jax.ShapeDtypeStruct((N_TOK, D_MODEL), jnp.float32),
    )(yp, yp)


def kernel(x, router_w, w_gate_up, w_down):
    Bb, Tt, D = x.shape
    x_flat = x.reshape(-1, D)

    pos0, pos1, perm, wsort, meta = _run_router(x_flat, router_w)

    perm_pad = perm.reshape(AP)
    w_pad = wsort.reshape(AP, 1)

    x_sorted = _sc_gather_rows(x_flat, perm_pad)

    wgu_bf = w_gate_up.astype(jnp.bfloat16)
    y_sorted = _run_mlp(meta, x_sorted, w_pad, wgu_bf, w_down)

    yp = _sc_gather_pairs(y_sorted, pos0.reshape(N_TOK), pos1.reshape(N_TOK))
    out = _run_pair_add(yp)
    return out.reshape(Bb, Tt, D)


# speedup vs baseline: 3.0319x; 1.0853x over previous
import functools

import jax
import jax.numpy as jnp
from jax import lax
from jax.experimental import pallas as pl
from jax.experimental.pallas import tpu as pltpu
from jax.experimental.pallas import tpu_sc as plsc

D_MODEL = 1024
D_FF = 2048
N_EXPERTS = 8
N_TOK = 2048
N_ASN = 2 * N_TOK
BM = 256
MAX_TILES = 24
AP = MAX_TILES * BM
DUMP_BLK = MAX_TILES - 1
NW = 32


def _router_body(x_ref, rw_ref, pos0_ref, pos1_ref, perm_ref, wsort_ref,
                 meta_ref):
    N, E, A = N_TOK, N_EXPERTS, N_ASN
    x = x_ref[...]
    rw = rw_ref[...]
    logits = lax.dot_general(x, rw, (((1,), (1,)), ((), ())),
                             preferred_element_type=jnp.float32)

    idx8 = lax.broadcasted_iota(jnp.int32, (N, E), 1)
    big_neg = jnp.float32(-3.0e38)
    m1 = jnp.max(logits, axis=1, keepdims=True)
    i1 = jnp.min(jnp.where(logits == m1, idx8, E), axis=1, keepdims=True)
    l2 = jnp.where(idx8 == i1, big_neg, logits)
    m2 = jnp.max(l2, axis=1, keepdims=True)
    i2 = jnp.min(jnp.where(l2 == m2, idx8, E), axis=1, keepdims=True)
    d = jnp.exp(m2 - m1)
    w1 = 1.0 / (1.0 + d)
    w2 = d / (1.0 + d)

    oh1 = (idx8 == i1).astype(jnp.float32)
    oh2 = (idx8 == i2).astype(jnp.float32)
    counts = (jnp.sum(oh1, axis=0, keepdims=True)
              + jnp.sum(oh2, axis=0, keepdims=True))
    cpad = (((counts.astype(jnp.int32) + (BM - 1)) // BM) * BM).astype(
        jnp.float32)
    lt8 = (lax.broadcasted_iota(jnp.int32, (E, E), 0)
           < lax.broadcasted_iota(jnp.int32, (E, E), 1)).astype(jnp.float32)
    offs = jnp.dot(cpad, lt8, preferred_element_type=jnp.float32)

    BL = 1024
    ltb = (lax.broadcasted_iota(jnp.int32, (BL, BL), 0)
           > lax.broadcasted_iota(jnp.int32, (BL, BL), 1)).astype(jnp.float32)
    base = jnp.zeros((1, E), jnp.float32)
    pos_blocks = []
    oh_blocks = [oh1[:BL], oh1[BL:], oh2[:BL], oh2[BL:]]
    for ohb in oh_blocks:
        rb = jnp.dot(ltb, ohb, preferred_element_type=jnp.float32) + base
        base = base + jnp.sum(ohb, axis=0, keepdims=True)
        pos_blocks.append(jnp.sum((rb + offs) * ohb, axis=1, keepdims=True))
    pos = jnp.concatenate(pos_blocks, axis=0)
    pos0_ref[...] = pos[:N].astype(jnp.int32)
    pos1_ref[...] = pos[N:].astype(jnp.int32)

    tok = lax.broadcasted_iota(jnp.int32, (N, 1), 0).astype(jnp.float32)
    tok_a = jnp.concatenate([tok, tok], axis=0)
    w_a = jnp.concatenate([w1, w2], axis=0)
    for pb in range(AP // 256):
        iota256 = lax.broadcasted_iota(jnp.int32, (1, 256), 1).astype(
            jnp.float32)
        pids = jnp.float32(pb * 256) + iota256
        m = pos == pids
        r, c = pb // 2, (pb % 2) * 256
        p1 = jnp.sum(jnp.where(m, tok_a + 1.0, 0.0), axis=0, keepdims=True)
        fill = jnp.float32((pb % 8) * 256) + iota256
        perm_ref[r:r + 1, c:c + 256] = jnp.where(
            p1 == 0.0, fill, p1 - 1.0).astype(jnp.int32)
        wsort_ref[r:r + 1, c:c + 256] = jnp.sum(
            jnp.where(m, w_a, 0.0), axis=0, keepdims=True)

    counts_i = counts.astype(jnp.int32)
    nt = (counts_i + (BM - 1)) // BM
    nt_f = nt.astype(jnp.float32)
    le8 = (lax.broadcasted_iota(jnp.int32, (E, E), 0)
           <= lax.broadcasted_iota(jnp.int32, (E, E), 1)).astype(jnp.float32)
    cumnt = jnp.dot(nt_f, le8, preferred_element_type=jnp.float32)
    tstart = cumnt - nt_f
    tid = lax.broadcasted_iota(jnp.int32, (MAX_TILES, 1), 0).astype(
        jnp.float32)
    te = jnp.sum((cumnt <= tid).astype(jnp.float32), axis=1, keepdims=True)
    te = jnp.minimum(te, jnp.float32(E - 1))
    idx8t = lax.broadcasted_iota(jnp.int32, (MAX_TILES, E), 1).astype(
        jnp.float32)
    oh_te = (idx8t == te).astype(jnp.float32)
    tstart_te = jnp.sum(oh_te * tstart, axis=1, keepdims=True)
    offs_te = jnp.sum(oh_te * offs, axis=1, keepdims=True)
    cnt_te = jnp.sum(oh_te * counts, axis=1, keepdims=True)
    j = tid - tstart_te
    rsb = offs_te / float(BM) + j
    nv = jnp.clip(cnt_te - j * BM, 0.0, float(BM))
    total = jnp.sum(nt_f)
    nv = jnp.where(tid < total, nv, 0.0)
    rsb = jnp.where(tid < total, rsb, float(DUMP_BLK))
    meta_ref[...] = jnp.concatenate([te, rsb, nv, tid],
                                    axis=1).astype(jnp.int32)


def _run_router(x_flat, router_w):
    return pl.pallas_call(
        _router_body,
        out_shape=(
            jax.ShapeDtypeStruct((N_TOK, 1), jnp.int32),
            jax.ShapeDtypeStruct((N_TOK, 1), jnp.int32),
            jax.ShapeDtypeStruct((AP // 512, 512), jnp.int32),
            jax.ShapeDtypeStruct((AP // 512, 512), jnp.float32),
            jax.ShapeDtypeStruct((MAX_TILES, 4), jnp.int32),
        ),
        compiler_params=pltpu.CompilerParams(
            vmem_limit_bytes=100 * 1024 * 1024),
    )(x_flat, router_w)


def _sc_gather_rows(x_flat, perm_pad):
    rows_w = AP // NW
    nch = rows_w // 32
    mesh = plsc.VectorSubcoreMesh(core_axis_name="c", subcore_axis_name="s")

    @functools.partial(
        pl.kernel,
        out_type=jax.ShapeDtypeStruct((AP, D_MODEL), jnp.float32),
        mesh=mesh,
        scratch_types=[
            pltpu.VMEM((rows_w,), jnp.int32),
            pltpu.VMEM((32, D_MODEL), jnp.float32),
            pltpu.VMEM((32, D_MODEL), jnp.float32),
            pltpu.VMEM((32, D_MODEL), jnp.float32),
            pltpu.SemaphoreType.DMA,
        ],
    )
    def gather_k(x_hbm, idx_hbm, out_hbm, idx_v, b0, b1, b2, sem):
        wid = lax.axis_index("s") * 2 + lax.axis_index("c")
        base = wid * rows_w
        bufs = [b0, b1, b2]
        pltpu.sync_copy(idx_hbm.at[pl.ds(base, rows_w)], idx_v)
        cps = [None] * nch
        for c in range(3):
            cps[c] = pltpu.async_copy(
                x_hbm.at[idx_v.at[pl.ds(c * 32, 32)]], bufs[c], sem)
        for c in range(nch):
            cps[c].wait()
            pltpu.sync_copy(bufs[c % 3], out_hbm.at[pl.ds(base + c * 32, 32)])
            if c + 3 < nch:
                cps[c + 3] = pltpu.async_copy(
                    x_hbm.at[idx_v.at[pl.ds((c + 3) * 32, 32)]], bufs[c % 3],
                    sem)

    return gather_k(x_flat, perm_pad)


def _mlp_body(meta_ref, x_ref, w_ref, wgu_ref, wd_ref, o_ref):
    t = pl.program_id(0)
    nv = meta_ref[t, 2]

    @pl.when(nv > 0)
    def _():
        xb = x_ref[...].astype(jnp.bfloat16)
        gu = lax.dot_general(xb, wgu_ref[0].astype(jnp.bfloat16),
                             (((1,), (0,)), ((), ())),
                             preferred_element_type=jnp.float32)
        g = gu[:, :D_FF]
        u = gu[:, D_FF:]
        inter = (g * (1.0 / (1.0 + jnp.exp(-g)))) * u
        interb = (inter * w_ref[...]).astype(jnp.bfloat16)
        o_ref[...] = lax.dot_general(
            interb, wd_ref[0], (((1,), (0,)), ((), ())),
            preferred_element_type=jnp.float32)


def _run_mlp(meta, x_sorted, w_pad, wgu_bf, wd_bf):
    grid_spec = pltpu.PrefetchScalarGridSpec(
        num_scalar_prefetch=1,
        grid=(MAX_TILES,),
        in_specs=[
            pl.BlockSpec((BM, D_MODEL), lambda t, m: (m[t, 1], 0)),
            pl.BlockSpec((BM, 1), lambda t, m: (m[t, 1], 0)),
            pl.BlockSpec((1, D_MODEL, 2 * D_FF), lambda t, m: (m[t, 0], 0, 0)),
            pl.BlockSpec((1, D_FF, D_MODEL), lambda t, m: (m[t, 0], 0, 0)),
        ],
        out_specs=pl.BlockSpec((BM, D_MODEL), lambda t, m: (m[t, 1], 0)),
    )
    return pl.pallas_call(
        _mlp_body,
        grid_spec=grid_spec,
        out_shape=jax.ShapeDtypeStruct((AP, D_MODEL), jnp.float32),
    )(meta, x_sorted, w_pad, wgu_bf, wd_bf)


def _sc_gather_pairs(y_sorted, pos0, pos1):
    tok_w = N_TOK // NW
    mesh = plsc.VectorSubcoreMesh(core_axis_name="c", subcore_axis_name="s")

    @functools.partial(
        pl.kernel,
        out_type=jax.ShapeDtypeStruct((2, N_TOK, D_MODEL), jnp.float32),
        mesh=mesh,
        scratch_types=[
            pltpu.VMEM((tok_w,), jnp.int32),
            pltpu.VMEM((tok_w,), jnp.int32),
            pltpu.VMEM((32, D_MODEL), jnp.float32),
            pltpu.VMEM((32, D_MODEL), jnp.float32),
            pltpu.SemaphoreType.DMA,
        ],
    )
    def pairs_k(y_hbm, p0_hbm, p1_hbm, out_hbm, i0_v, i1_v, b0, b1, sem):
        wid = lax.axis_index("s") * 2 + lax.axis_index("c")
        base = wid * tok_w
        pltpu.sync_copy(p0_hbm.at[pl.ds(base, tok_w)], i0_v)
        pltpu.sync_copy(p1_hbm.at[pl.ds(base, tok_w)], i1_v)
        for c in range(tok_w // 32):
            cp0 = pltpu.async_copy(
                y_hbm.at[i0_v.at[pl.ds(c * 32, 32)]], b0, sem)
            cp1 = pltpu.async_copy(
                y_hbm.at[i1_v.at[pl.ds(c * 32, 32)]], b1, sem)
            cp0.wait()
            pltpu.sync_copy(b0, out_hbm.at[0, pl.ds(base + c * 32, 32)])
            cp1.wait()
            pltpu.sync_copy(b1, out_hbm.at[1, pl.ds(base + c * 32, 32)])

    return pairs_k(y_sorted, pos0, pos1)


def _add_body(a_ref, b_ref, o_ref):
    o_ref[...] = a_ref[0] + b_ref[0]


def _run_pair_add(yp):
    return pl.pallas_call(
        _add_body,
        grid=(N_TOK // BM,),
        in_specs=[
            pl.BlockSpec((1, BM, D_MODEL), lambda m: (0, m, 0)),
            pl.BlockSpec((1, BM, D_MODEL), lambda m: (1, m, 0)),
        ],
        out_specs=pl.BlockSpec((BM, D_MODEL), lambda m: (m, 0)),
        out_shape=jax.ShapeDtypeStruct((N_TOK, D_MODEL), jnp.float32),
    )(yp, yp)


def kernel(x, router_w, w_gate_up, w_down):
    Bb, Tt, D = x.shape
    x_flat = x.reshape(-1, D)

    pos0, pos1, perm, wsort, meta = _run_router(x_flat, router_w)

    perm_pad = perm.reshape(AP)
    w_pad = wsort.reshape(AP, 1)

    x_sorted = _sc_gather_rows(x_flat, perm_pad)

    wd_bf = w_down.astype(jnp.bfloat16)
    y_sorted = _run_mlp(meta, x_sorted, w_pad, w_gate_up, wd_bf)

    yp = _sc_gather_pairs(y_sorted, pos0.reshape(N_TOK), pos1.reshape(N_TOK))
    out = _run_pair_add(yp)
    return out.reshape(Bb, Tt, D)
